# baseline (speedup 1.0000x reference)
import jax
import jax.numpy as jnp
from jax.experimental import pallas as pl

B, LATENT, T = 8, 1024, 2048
D, K = 64, 1024


def _vq_kernel(x_ref, v_in_ref, g_in_ref, b_in_ref, cb_ref,
               v_out_ref, g_out_ref, b_out_ref,
               out_ref, loss1_ref, loss2_ref, idx_ref, proj_ref):
    pid = pl.program_id(0)

    v_in = v_in_ref[...]
    g_in = g_in_ref[...].reshape(D, 1)
    w_in = v_in * (g_in / jnp.sqrt(jnp.sum(v_in * v_in, axis=1, keepdims=True)))
    v_out = v_out_ref[...]
    g_out = g_out_ref[...].reshape(LATENT, 1)
    w_out = v_out * (g_out / jnp.sqrt(jnp.sum(v_out * v_out, axis=1, keepdims=True)))
    cb = cb_ref[...]
    cbn = cb / jnp.maximum(jnp.sqrt(jnp.sum(cb * cb, axis=1, keepdims=True)), 1e-12)
    csq = jnp.sum(cbn * cbn, axis=1, keepdims=True)

    x = x_ref[0]
    p = jax.lax.dot_general(w_in, x, (((1,), (0,)), ((), ())))
    p = p + b_in_ref[...].reshape(D, 1)

    norm = jnp.sqrt(jnp.sum(p * p, axis=0, keepdims=True))
    en = p / jnp.maximum(norm, 1e-12)
    l2 = jnp.sum(en * en, axis=0, keepdims=True)
    s = jax.lax.dot_general(cbn, en, (((1,), (0,)), ((), ())))
    dist = l2 - 2.0 * s + csq

    idx = jnp.argmin(dist, axis=0)
    brow = jax.lax.broadcasted_iota(jnp.int32, (B, T), 0)
    idx_ref[...] = jnp.where(brow == pid, idx[None, :], idx_ref[...])

    iota = jax.lax.broadcasted_iota(jnp.int32, dist.shape, 0)
    oh = (iota == idx[None, :]).astype(jnp.bfloat16)
    q = jax.lax.dot_general(cb, oh, (((0,), (0,)), ((), ())),
                            preferred_element_type=jnp.float32)

    proj_ref[0] = p

    loss = jnp.sum((p - q) ** 2) / (D * T)
    blane = jax.lax.broadcasted_iota(jnp.int32, (B,), 0)
    loss1_ref[...] = jnp.where(blane == pid, loss, loss1_ref[...])
    loss2_ref[...] = jnp.where(blane == pid, loss, loss2_ref[...])

    out = jax.lax.dot_general(w_out, q, (((1,), (0,)), ((), ())))
    out_ref[0] = out + b_out_ref[...].reshape(LATENT, 1)


def kernel(hidden_state, v_in, g_in, b_in, codebook, v_out, g_out, b_out):
    out_shapes = (
        jax.ShapeDtypeStruct((B, LATENT, T), jnp.float32),
        jax.ShapeDtypeStruct((B,), jnp.float32),
        jax.ShapeDtypeStruct((B,), jnp.float32),
        jax.ShapeDtypeStruct((B, T), jnp.int32),
        jax.ShapeDtypeStruct((B, D, T), jnp.float32),
    )
    return pl.pallas_call(
        _vq_kernel,
        grid=(B,),
        in_specs=[
            pl.BlockSpec((1, LATENT, T), lambda b: (b, 0, 0)),
            pl.BlockSpec((D, LATENT), lambda b: (0, 0)),
            pl.BlockSpec((D,), lambda b: (0,)),
            pl.BlockSpec((D,), lambda b: (0,)),
            pl.BlockSpec((K, D), lambda b: (0, 0)),
            pl.BlockSpec((LATENT, D), lambda b: (0, 0)),
            pl.BlockSpec((LATENT,), lambda b: (0,)),
            pl.BlockSpec((LATENT,), lambda b: (0,)),
        ],
        out_specs=(
            pl.BlockSpec((1, LATENT, T), lambda b: (b, 0, 0)),
            pl.BlockSpec((B,), lambda b: (0,)),
            pl.BlockSpec((B,), lambda b: (0,)),
            pl.BlockSpec((B, T), lambda b: (0, 0)),
            pl.BlockSpec((1, D, T), lambda b: (b, 0, 0)),
        ),
        out_shape=out_shapes,
    )(hidden_state, v_in, g_in, b_in, codebook, v_out, g_out, b_out)
